# baseline (speedup 1.0000x reference)
import functools

import jax
import jax.numpy as jnp
import numpy as np
from jax import lax
from jax.experimental import pallas as pl
from jax.experimental.pallas import tpu as pltpu
from jax.experimental.pallas import tpu_sc as plsc

N = 10000
E = 320000
D = 128
H = 4
EMB = 8
HID = 64

NC = 2
NS = 16
NW = NC * NS
MICRO = 80
CH = 5
EC = E // CH
RPC = EC // (NW * MICRO)
STRIPE = 632
NPAD = NS * STRIPE

BE = 1280
BN = 400

_HIGH = lax.Precision.HIGHEST
NBUF = 4


def _gather_body(nodef, src2, dst2, xs, xd, idx_s, idx_d, rows_a, rows_b,
                 sem_a, sem_b):
    c = lax.axis_index("c")
    s = lax.axis_index("s")
    wid = s * NC + c
    rowbase = wid * RPC
    pltpu.sync_copy(src2.at[wid], idx_s)
    pltpu.sync_copy(dst2.at[wid], idx_d)

    for b in range(NBUF - 1):
        pltpu.async_copy(nodef.at[idx_s.at[b]], rows_a.at[b], sem_a.at[b])
        pltpu.async_copy(nodef.at[idx_d.at[b]], rows_b.at[b], sem_b.at[b])

    def body(j, carry):
        p = lax.rem(j, NBUF)
        jn = j + NBUF - 1
        pn = lax.rem(jn, NBUF)

        @pl.when(jn < RPC)
        def _():
            pltpu.async_copy(nodef.at[idx_s.at[jn]], rows_a.at[pn],
                             sem_a.at[pn])
            pltpu.async_copy(nodef.at[idx_d.at[jn]], rows_b.at[pn],
                             sem_b.at[pn])

        e0 = (rowbase + j) * MICRO
        pltpu.make_async_copy(nodef.at[idx_s.at[j]], rows_a.at[p],
                              sem_a.at[p]).wait()
        pltpu.sync_copy(rows_a.at[p], xs.at[pl.ds(e0, MICRO)])
        pltpu.make_async_copy(nodef.at[idx_d.at[j]], rows_b.at[p],
                              sem_b.at[p]).wait()
        pltpu.sync_copy(rows_b.at[p], xd.at[pl.ds(e0, MICRO)])
        return carry

    lax.fori_loop(0, RPC, body, 0)


def _sc_gather(node_f, src2, dst2):
    mesh = plsc.VectorSubcoreMesh(core_axis_name="c", subcore_axis_name="s")
    k = functools.partial(
        pl.kernel,
        mesh=mesh,
        out_type=[
            jax.ShapeDtypeStruct((EC, D), jnp.float32),
            jax.ShapeDtypeStruct((EC, D), jnp.float32),
        ],
        scratch_types=[
            pltpu.VMEM((RPC, MICRO), jnp.int32),
            pltpu.VMEM((RPC, MICRO), jnp.int32),
            pltpu.VMEM((NBUF, MICRO, D), jnp.float32),
            pltpu.VMEM((NBUF, MICRO, D), jnp.float32),
            pltpu.SemaphoreType.DMA((NBUF,)),
            pltpu.SemaphoreType.DMA((NBUF,)),
        ],
    )(_gather_body)
    return k(node_f, src2, dst2)


def _scatter_body(*refs):
    dsts = refs[0:CH]
    expvs = refs[CH:2 * CH]
    evps = refs[2 * CH:3 * CH]
    zset, noset, z2, no2, idx_d, expbuf, evbuf, zsh, nosh = refs[3 * CH:]
    c = lax.axis_index("c")
    s = lax.axis_index("s")
    wid = s * NC + c
    rowbase = wid * RPC
    n0 = s * STRIPE
    pltpu.sync_copy(zset.at[pl.ds(n0, STRIPE)], zsh.at[pl.ds(n0, STRIPE)])
    pltpu.sync_copy(noset.at[pl.ds(n0, STRIPE)], nosh.at[pl.ds(n0, STRIPE)])
    plsc.subcore_barrier()

    for t in range(CH):
        expv, evp = expvs[t], evps[t]
        pltpu.sync_copy(dsts[t].at[wid], idx_d)

        def body(j, carry):
            e0 = (rowbase + j) * MICRO
            pltpu.sync_copy(expv.at[pl.ds(e0, MICRO)], expbuf)
            pltpu.sync_copy(evp.at[pl.ds(e0, MICRO)], evbuf)
            pltpu.sync_copy(expbuf, zsh.at[idx_d.at[j]], add=True)
            pltpu.sync_copy(evbuf, nosh.at[idx_d.at[j]], add=True)
            return carry

        lax.fori_loop(0, RPC, body, 0)

    plsc.subcore_barrier()
    pltpu.sync_copy(zsh.at[pl.ds(n0, STRIPE)], z2.at[c, pl.ds(n0, STRIPE)])
    pltpu.sync_copy(nosh.at[pl.ds(n0, STRIPE)], no2.at[c, pl.ds(n0, STRIPE)])


def _sc_scatter(dsts, expvs, evps, zset, noset):
    mesh = plsc.VectorSubcoreMesh(core_axis_name="c", subcore_axis_name="s")
    k = functools.partial(
        pl.kernel,
        mesh=mesh,
        out_type=[
            jax.ShapeDtypeStruct((NC, NPAD, 8), jnp.float32),
            jax.ShapeDtypeStruct((NC, NPAD, D), jnp.float32),
        ],
        scratch_types=[
            pltpu.VMEM((RPC, MICRO), jnp.int32),
            pltpu.VMEM((MICRO, 8), jnp.float32),
            pltpu.VMEM((MICRO, D), jnp.float32),
            pltpu.VMEM_SHARED((NPAD, 8), jnp.float32),
            pltpu.VMEM_SHARED((NPAD, D), jnp.float32),
        ],
    )(_scatter_body)
    return k(*dsts, *expvs, *evps, zset, noset)


def _edge_body(emb_ref, cut_ref, attr_ref, xs_ref, xd_ref, w1_ref, w2_ref,
               w3_ref, wq_ref, ered_ref, exp_ref, expv_ref, evp_ref):
    bf = jnp.bfloat16
    f32 = jnp.float32
    emb = emb_ref[...].astype(bf)
    h1 = jnp.maximum(jnp.dot(emb, w1_ref[...],
                             preferred_element_type=f32), 0.0)
    h2 = jnp.maximum(jnp.dot(h1.astype(bf), w2_ref[...],
                             preferred_element_type=f32), 0.0)
    wc = jnp.dot(h2.astype(bf), w3_ref[...], preferred_element_type=f32)
    wck = wc[:, :D]
    wcv = wc[:, D:]

    xsa = xs_ref[...] * attr_ref[...]
    xd = xd_ref[...]
    ek = wck * xsa
    q = jnp.dot(ek.astype(bf), wq_ref[...],
                preferred_element_type=f32)
    xdt = jnp.concatenate([xd, xd, xd, xd], axis=1)
    dot8 = jnp.dot((q * xdt).astype(bf), ered_ref[...],
                   preferred_element_type=f32)
    expv = cut_ref[...] * jnp.exp(dot8)
    expv_ref[...] = expv
    s8 = jnp.sqrt(jnp.maximum(expv, 0.0)).astype(bf)
    sexp = jnp.dot(s8, exp_ref[...], preferred_element_type=f32)
    evp_ref[...] = wcv * xsa * sexp


def _tc_edge(emb, cut, attr, xs, xd, w1, w2, w3, wq, ered, expand8):
    grid = (EC // BE,)
    full = lambda shape: pl.BlockSpec(shape, lambda i: (0, 0))
    return pl.pallas_call(
        _edge_body,
        grid=grid,
        in_specs=[
            pl.BlockSpec((BE, EMB), lambda i: (i, 0)),
            pl.BlockSpec((BE, 1), lambda i: (i, 0)),
            pl.BlockSpec((BE, 1), lambda i: (i, 0)),
            pl.BlockSpec((BE, D), lambda i: (i, 0)),
            pl.BlockSpec((BE, D), lambda i: (i, 0)),
            full((EMB, 2 * HID)),
            full((2 * HID, 2 * HID)),
            full((2 * HID, 2 * D)),
            full((D, H * D)),
            full((H * D, 8)),
            full((8, D)),
        ],
        out_specs=[
            pl.BlockSpec((BE, 8), lambda i: (i, 0)),
            pl.BlockSpec((BE, D), lambda i: (i, 0)),
        ],
        out_shape=[
            jax.ShapeDtypeStruct((EC, 8), jnp.float32),
            jax.ShapeDtypeStruct((EC, D), jnp.float32),
        ],
    )(emb, cut, attr, xs, xd, w1, w2, w3, wq, ered, expand8)


def _final_body(z2_ref, no2_ref, wlin_ref, exp_ref, out_ref):
    z = z2_ref[0] + z2_ref[1]
    z = jnp.where(z == 0.0, 1.0, z)
    rs = lax.rsqrt(z)
    zfac = jnp.dot(rs, exp_ref[...], precision=_HIGH)
    no = no2_ref[0] + no2_ref[1]
    out_ref[...] = jnp.dot(no * zfac, wlin_ref[...], precision=_HIGH)


def _tc_final(z2, no2, wlin, expand8):
    grid = (N // BN,)
    return pl.pallas_call(
        _final_body,
        grid=grid,
        in_specs=[
            pl.BlockSpec((NC, BN, 8), lambda i: (0, i, 0)),
            pl.BlockSpec((NC, BN, D), lambda i: (0, i, 0)),
            pl.BlockSpec((D, D), lambda i: (0, 0)),
            pl.BlockSpec((8, D), lambda i: (0, 0)),
        ],
        out_specs=pl.BlockSpec((BN, D), lambda i: (i, 0)),
        out_shape=jax.ShapeDtypeStruct((N, D), jnp.float32),
    )(z2, no2, wlin, expand8)


def kernel(edge_src, edge_dst, edge_scalar_attr, edge_attr, edge_weight_cutoff,
           node_f, W1k, W2k, wk, Wdot, W1v, W2v, wv, Wlin):
    bf = jnp.bfloat16
    w1 = (jnp.concatenate([W1k, W1v], axis=1) / np.sqrt(EMB)).astype(bf)
    zh = jnp.zeros((HID, HID), jnp.float32)
    w2 = (jnp.block([[W2k, zh], [zh, W2v]]) / np.sqrt(HID)).astype(bf)
    zd = jnp.zeros((HID, D), jnp.float32)
    w3 = (jnp.block([[wk[:, :, 0], zd], [zd, wv[:, :, 0]]]) / np.sqrt(HID)).astype(bf)
    wq = (jnp.transpose(Wdot, (2, 0, 1)).reshape(D, H * D) / D).astype(bf)
    wlin = Wlin / np.sqrt(D)
    expand8 = (jnp.arange(8, dtype=jnp.int32)[:, None]
               == (jnp.arange(D, dtype=jnp.int32)[None, :] // (D // H))
               ).astype(jnp.float32)
    ered = (jnp.arange(H * D, dtype=jnp.int32)[:, None] // D
            == jnp.arange(8, dtype=jnp.int32)[None, :]).astype(bf)
    e8b = expand8.astype(bf)

    cut = edge_weight_cutoff.reshape(E, 1)
    zset = jnp.zeros((NPAD, 8), jnp.float32)
    noset = jnp.zeros((NPAD, D), jnp.float32)

    dsts, expvs, evps = [], [], []
    for t in range(CH):
        lo = t * EC
        src_c = lax.dynamic_slice_in_dim(edge_src, lo, EC).reshape(NW, RPC, MICRO)
        dst_c = lax.dynamic_slice_in_dim(edge_dst, lo, EC).reshape(NW, RPC, MICRO)
        xs, xd = _sc_gather(node_f, src_c, dst_c)
        expv, evp = _tc_edge(
            lax.dynamic_slice_in_dim(edge_scalar_attr, lo, EC),
            lax.dynamic_slice_in_dim(cut, lo, EC),
            lax.dynamic_slice_in_dim(edge_attr, lo, EC),
            xs, xd, w1, w2, w3, wq, ered, e8b)
        dsts.append(dst_c)
        expvs.append(expv)
        evps.append(evp)

    z2, no2 = _sc_scatter(dsts, expvs, evps, zset, noset)
    return _tc_final(z2, no2, wlin, expand8)


# speedup vs baseline: 1.1775x; 1.1775x over previous
import functools

import jax
import jax.numpy as jnp
import numpy as np
from jax import lax
from jax.experimental import pallas as pl
from jax.experimental.pallas import tpu as pltpu
from jax.experimental.pallas import tpu_sc as plsc

N = 10000
E = 320000
D = 128
H = 4
EMB = 8
HID = 64

NC = 2
NS = 16
NW = NC * NS
MICRO = 80
CH = 1
EC = E // CH
RPC = EC // (NW * MICRO)
STRIPE = 632
NPAD = NS * STRIPE

BE = 1280
BN = 400

_HIGH = lax.Precision.HIGHEST
NBUF = 4


def _gather_body(nodef, src2, dst2, xs, xd, idx_s, idx_d, rows_a, rows_b,
                 sem_a, sem_b):
    c = lax.axis_index("c")
    s = lax.axis_index("s")
    wid = s * NC + c
    rowbase = wid * RPC
    pltpu.sync_copy(src2.at[wid], idx_s)
    pltpu.sync_copy(dst2.at[wid], idx_d)

    for b in range(NBUF - 1):
        pltpu.async_copy(nodef.at[idx_s.at[b]], rows_a.at[b], sem_a.at[b])
        pltpu.async_copy(nodef.at[idx_d.at[b]], rows_b.at[b], sem_b.at[b])

    def body(j, carry):
        p = lax.rem(j, NBUF)
        jn = j + NBUF - 1
        pn = lax.rem(jn, NBUF)

        @pl.when(jn < RPC)
        def _():
            pltpu.async_copy(nodef.at[idx_s.at[jn]], rows_a.at[pn],
                             sem_a.at[pn])
            pltpu.async_copy(nodef.at[idx_d.at[jn]], rows_b.at[pn],
                             sem_b.at[pn])

        e0 = (rowbase + j) * MICRO
        pltpu.make_async_copy(nodef.at[idx_s.at[j]], rows_a.at[p],
                              sem_a.at[p]).wait()
        pltpu.sync_copy(rows_a.at[p], xs.at[pl.ds(e0, MICRO)])
        pltpu.make_async_copy(nodef.at[idx_d.at[j]], rows_b.at[p],
                              sem_b.at[p]).wait()
        pltpu.sync_copy(rows_b.at[p], xd.at[pl.ds(e0, MICRO)])
        return carry

    lax.fori_loop(0, RPC, body, 0)


def _sc_gather(node_f, src2, dst2):
    mesh = plsc.VectorSubcoreMesh(core_axis_name="c", subcore_axis_name="s")
    k = functools.partial(
        pl.kernel,
        mesh=mesh,
        out_type=[
            jax.ShapeDtypeStruct((EC, D), jnp.float32),
            jax.ShapeDtypeStruct((EC, D), jnp.float32),
        ],
        scratch_types=[
            pltpu.VMEM((RPC, MICRO), jnp.int32),
            pltpu.VMEM((RPC, MICRO), jnp.int32),
            pltpu.VMEM((NBUF, MICRO, D), jnp.float32),
            pltpu.VMEM((NBUF, MICRO, D), jnp.float32),
            pltpu.SemaphoreType.DMA((NBUF,)),
            pltpu.SemaphoreType.DMA((NBUF,)),
        ],
    )(_gather_body)
    return k(node_f, src2, dst2)


def _make_scatter_body(width):
    def body_fn(dst2, vals, init, out, idx_d, vbuf, acc, sem_v):
        c = lax.axis_index("c")
        s = lax.axis_index("s")
        wid = s * NC + c
        rowbase = wid * RPC
        n0 = s * STRIPE
        pltpu.sync_copy(init.at[pl.ds(n0, STRIPE)], acc.at[pl.ds(n0, STRIPE)])
        pltpu.sync_copy(dst2.at[wid], idx_d)
        plsc.subcore_barrier()
        pltpu.async_copy(vals.at[pl.ds(rowbase * MICRO, MICRO)], vbuf.at[0],
                         sem_v.at[0])

        def body(j, carry):
            p = lax.rem(j, 2)
            jn = j + 1
            pn = lax.rem(jn, 2)
            e0 = (rowbase + j) * MICRO

            @pl.when(jn < RPC)
            def _():
                en = (rowbase + jn) * MICRO
                pltpu.async_copy(vals.at[pl.ds(en, MICRO)], vbuf.at[pn],
                                 sem_v.at[pn])

            pltpu.make_async_copy(vals.at[pl.ds(e0, MICRO)], vbuf.at[p],
                                  sem_v.at[p]).wait()
            pltpu.sync_copy(vbuf.at[p], acc.at[idx_d.at[j]], add=True)
            return carry

        lax.fori_loop(0, RPC, body, 0)
        plsc.subcore_barrier()
        pltpu.sync_copy(acc.at[pl.ds(n0, STRIPE)],
                        out.at[c, pl.ds(n0, STRIPE)])

    return body_fn


def _sc_scatter_one(dst2, vals, init, width):
    mesh = plsc.VectorSubcoreMesh(core_axis_name="c", subcore_axis_name="s")
    k = functools.partial(
        pl.kernel,
        mesh=mesh,
        out_type=jax.ShapeDtypeStruct((NC, NPAD, width), jnp.float32),
        scratch_types=[
            pltpu.VMEM((RPC, MICRO), jnp.int32),
            pltpu.VMEM((2, MICRO, width), jnp.float32),
            pltpu.VMEM_SHARED((NPAD, width), jnp.float32),
            pltpu.SemaphoreType.DMA((2,)),
        ],
    )(_make_scatter_body(width))
    return k(dst2, vals, init)


def _edge_body(emb_ref, cut_ref, attr_ref, xs_ref, xd_ref, w1_ref, w2_ref,
               w3_ref, wq_ref, ered_ref, exp_ref, expv_ref, evp_ref):
    bf = jnp.bfloat16
    f32 = jnp.float32
    emb = emb_ref[...].astype(bf)
    h1 = jnp.maximum(jnp.dot(emb, w1_ref[...],
                             preferred_element_type=f32), 0.0)
    h2 = jnp.maximum(jnp.dot(h1.astype(bf), w2_ref[...],
                             preferred_element_type=f32), 0.0)
    wc = jnp.dot(h2.astype(bf), w3_ref[...], preferred_element_type=f32)
    wck = wc[:, :D]
    wcv = wc[:, D:]

    xsa = xs_ref[...] * attr_ref[...]
    xd = xd_ref[...]
    ek = wck * xsa
    q = jnp.dot(ek.astype(bf), wq_ref[...],
                preferred_element_type=f32)
    xdt = jnp.concatenate([xd, xd, xd, xd], axis=1)
    dot8 = jnp.dot((q * xdt).astype(bf), ered_ref[...],
                   preferred_element_type=f32)
    expv = cut_ref[...] * jnp.exp(dot8)
    expv_ref[...] = expv
    s8 = jnp.sqrt(jnp.maximum(expv, 0.0)).astype(bf)
    sexp = jnp.dot(s8, exp_ref[...], preferred_element_type=f32)
    evp_ref[...] = wcv * xsa * sexp


def _tc_edge(emb, cut, attr, xs, xd, w1, w2, w3, wq, ered, expand8):
    grid = (EC // BE,)
    full = lambda shape: pl.BlockSpec(shape, lambda i: (0, 0))
    return pl.pallas_call(
        _edge_body,
        grid=grid,
        in_specs=[
            pl.BlockSpec((BE, EMB), lambda i: (i, 0)),
            pl.BlockSpec((BE, 1), lambda i: (i, 0)),
            pl.BlockSpec((BE, 1), lambda i: (i, 0)),
            pl.BlockSpec((BE, D), lambda i: (i, 0)),
            pl.BlockSpec((BE, D), lambda i: (i, 0)),
            full((EMB, 2 * HID)),
            full((2 * HID, 2 * HID)),
            full((2 * HID, 2 * D)),
            full((D, H * D)),
            full((H * D, 8)),
            full((8, D)),
        ],
        out_specs=[
            pl.BlockSpec((BE, 8), lambda i: (i, 0)),
            pl.BlockSpec((BE, D), lambda i: (i, 0)),
        ],
        out_shape=[
            jax.ShapeDtypeStruct((EC, 8), jnp.float32),
            jax.ShapeDtypeStruct((EC, D), jnp.float32),
        ],
    )(emb, cut, attr, xs, xd, w1, w2, w3, wq, ered, expand8)


def _final_body(z2_ref, no2_ref, wlin_ref, exp_ref, out_ref):
    z = z2_ref[0] + z2_ref[1]
    z = jnp.where(z == 0.0, 1.0, z)
    rs = lax.rsqrt(z)
    zfac = jnp.dot(rs, exp_ref[...], precision=_HIGH)
    no = no2_ref[0] + no2_ref[1]
    out_ref[...] = jnp.dot(no * zfac, wlin_ref[...], precision=_HIGH)


def _tc_final(z2, no2, wlin, expand8):
    grid = (N // BN,)
    return pl.pallas_call(
        _final_body,
        grid=grid,
        in_specs=[
            pl.BlockSpec((NC, BN, 8), lambda i: (0, i, 0)),
            pl.BlockSpec((NC, BN, D), lambda i: (0, i, 0)),
            pl.BlockSpec((D, D), lambda i: (0, 0)),
            pl.BlockSpec((8, D), lambda i: (0, 0)),
        ],
        out_specs=pl.BlockSpec((BN, D), lambda i: (i, 0)),
        out_shape=jax.ShapeDtypeStruct((N, D), jnp.float32),
    )(z2, no2, wlin, expand8)


def kernel(edge_src, edge_dst, edge_scalar_attr, edge_attr, edge_weight_cutoff,
           node_f, W1k, W2k, wk, Wdot, W1v, W2v, wv, Wlin):
    bf = jnp.bfloat16
    w1 = (jnp.concatenate([W1k, W1v], axis=1) / np.sqrt(EMB)).astype(bf)
    zh = jnp.zeros((HID, HID), jnp.float32)
    w2 = (jnp.block([[W2k, zh], [zh, W2v]]) / np.sqrt(HID)).astype(bf)
    zd = jnp.zeros((HID, D), jnp.float32)
    w3 = (jnp.block([[wk[:, :, 0], zd], [zd, wv[:, :, 0]]]) / np.sqrt(HID)).astype(bf)
    wq = (jnp.transpose(Wdot, (2, 0, 1)).reshape(D, H * D) / D).astype(bf)
    wlin = Wlin / np.sqrt(D)
    expand8 = (jnp.arange(8, dtype=jnp.int32)[:, None]
               == (jnp.arange(D, dtype=jnp.int32)[None, :] // (D // H))
               ).astype(jnp.float32)
    ered = (jnp.arange(H * D, dtype=jnp.int32)[:, None] // D
            == jnp.arange(8, dtype=jnp.int32)[None, :]).astype(bf)
    e8b = expand8.astype(bf)

    cut = edge_weight_cutoff.reshape(E, 1)
    zset = jnp.zeros((NPAD, 8), jnp.float32)
    noset = jnp.zeros((NPAD, D), jnp.float32)

    src2 = edge_src.reshape(NW, RPC, MICRO)
    dst2 = edge_dst.reshape(NW, RPC, MICRO)
    xs, xd = _sc_gather(node_f, src2, dst2)
    expv, evp = _tc_edge(edge_scalar_attr, cut, edge_attr, xs, xd,
                         w1, w2, w3, wq, ered, e8b)
    no2 = _sc_scatter_one(dst2, evp, noset, D)
    z2 = _sc_scatter_one(dst2, expv, zset, 8)
    return _tc_final(z2, no2, wlin, expand8)
